# baseline (speedup 1.0000x reference)
import jax
import jax.numpy as jnp
from jax import lax
from jax.experimental import pallas as pl
from jax.experimental.pallas import tpu as pltpu
from jax.experimental.pallas import tpu_sc as plsc

L = 16
NS = 16
NW = NS
B = 16384
D = 64
BPW = B // NW
CH = 128
NCHUNK = BPW // CH
PASS = 512
CPP = PASS // CH


def _mf_body(ut, it, ubt, ibt, uid, iid, out,
             uid_v, iid_v, urows, irows, ub_v, ib_v, out_v, sem):
    wid = lax.axis_index("s")
    base = wid * BPW

    for j in range(NCHUNK):
        pltpu.sync_copy(uid.at[pl.ds(base + j * CH, CH)], uid_v.at[j])
        pltpu.sync_copy(iid.at[pl.ds(base + j * CH, CH)], iid_v.at[j])

    bias_copies = []
    for j in range(NCHUNK):
        bias_copies.append(pltpu.async_copy(
            ubt.at[uid_v.at[j]], ub_v.at[pl.ds(j * CH, CH)], sem))
        bias_copies.append(pltpu.async_copy(
            ibt.at[iid_v.at[j]], ib_v.at[pl.ds(j * CH, CH)], sem))

    lane = lax.iota(jnp.int32, L)

    for p in range(BPW // PASS):
        pb = p * PASS
        row_copies = []
        for q in range(CPP):
            j = p * CPP + q
            row_copies.append(pltpu.async_copy(
                ut.at[uid_v.at[j]], urows.at[pl.ds(q * CH, CH)], sem))
            row_copies.append(pltpu.async_copy(
                it.at[iid_v.at[j]], irows.at[pl.ds(q * CH, CH)], sem))
        if p == 0:
            for c in bias_copies:
                c.wait()
        for c in row_copies:
            c.wait()

        def group(g, carry):
            gb = pl.multiple_of(g * L, L)
            rows = gb + lane
            acc = ub_v[pl.ds(pb + gb, L)] + ib_v[pl.ds(pb + gb, L)]
            for k in range(D):
                col = lax.bitwise_and(lane + k, D - 1)
                u = plsc.load_gather(urows, [rows, col])
                v = plsc.load_gather(irows, [rows, col])
                acc = acc + u * v
            out_v[pl.ds(pb + gb, L)] = acc
            return carry

        lax.fori_loop(0, PASS // L, group, 0)

    pltpu.sync_copy(out_v, out.at[pl.ds(base, BPW)])


@jax.jit
def _mf(user_table, item_table, ub_flat, ib_flat, user_ids, item_ids):
    mesh = plsc.VectorSubcoreMesh(core_axis_name="c", subcore_axis_name="s",
                                  num_cores=1)
    kern = pl.kernel(
        _mf_body,
        mesh=mesh,
        compiler_params=pltpu.CompilerParams(use_tc_tiling_on_sc=False,
                                             needs_layout_passes=False),
        out_type=jax.ShapeDtypeStruct((B,), jnp.float32),
        scratch_types=[
            pltpu.VMEM((NCHUNK, CH), jnp.int32),
            pltpu.VMEM((NCHUNK, CH), jnp.int32),
            pltpu.VMEM((PASS, D), jnp.float32),
            pltpu.VMEM((PASS, D), jnp.float32),
            pltpu.VMEM((BPW,), jnp.float32),
            pltpu.VMEM((BPW,), jnp.float32),
            pltpu.VMEM((BPW,), jnp.float32),
            pltpu.SemaphoreType.DMA,
        ],
    )
    return kern(user_table, item_table, ub_flat, ib_flat, user_ids, item_ids)


def kernel(user_table, item_table, user_bias_table, item_bias_table,
           global_bias, user_ids, item_ids):
    out = _mf(user_table, item_table,
              user_bias_table.reshape(-1), item_bias_table.reshape(-1),
              user_ids.astype(jnp.int32), item_ids.astype(jnp.int32))
    return out + global_bias[0]


# speedup vs baseline: 1.0098x; 1.0098x over previous
import jax
import jax.numpy as jnp
from jax import lax
from jax.experimental import pallas as pl
from jax.experimental.pallas import tpu as pltpu
from jax.experimental.pallas import tpu_sc as plsc

L = 16
NC = 2
NS = 16
NW = NC * NS
B = 16384
D = 64
BPW = B // NW
CH = 128
NCHUNK = BPW // CH
GROUPS = BPW // L


def _mf_body(ut, it, ubt, ibt, uid, iid, out,
             uid_v, iid_v, urows, irows, ub_v, ib_v, out_v, sem):
    wid = lax.axis_index("s") * NC + lax.axis_index("c")
    base = wid * BPW

    for j in range(NCHUNK):
        pltpu.sync_copy(uid.at[pl.ds(base + j * CH, CH)], uid_v.at[j])
        pltpu.sync_copy(iid.at[pl.ds(base + j * CH, CH)], iid_v.at[j])

    copies = []
    for j in range(NCHUNK):
        copies.append(pltpu.async_copy(
            ut.at[uid_v.at[j]], urows.at[pl.ds(j * CH, CH)], sem))
        copies.append(pltpu.async_copy(
            it.at[iid_v.at[j]], irows.at[pl.ds(j * CH, CH)], sem))
        copies.append(pltpu.async_copy(
            ubt.at[uid_v.at[j]], ub_v.at[pl.ds(j * CH, CH)], sem))
        copies.append(pltpu.async_copy(
            ibt.at[iid_v.at[j]], ib_v.at[pl.ds(j * CH, CH)], sem))
    for c in copies:
        c.wait()

    lane = lax.iota(jnp.int32, L)

    def group(g, carry):
        gb = pl.multiple_of(g * L, L)
        rows = gb + lane
        acc = ub_v[pl.ds(gb, L)] + ib_v[pl.ds(gb, L)]
        for k in range(D):
            col = lax.bitwise_and(lane + k, D - 1)
            u = plsc.load_gather(urows, [rows, col])
            v = plsc.load_gather(irows, [rows, col])
            acc = acc + u * v
        out_v[pl.ds(gb, L)] = acc
        return carry

    lax.fori_loop(0, GROUPS, group, 0)
    pltpu.sync_copy(out_v, out.at[pl.ds(base, BPW)])


@jax.jit
def _mf(user_table, item_table, ub_flat, ib_flat, user_ids, item_ids):
    mesh = plsc.VectorSubcoreMesh(core_axis_name="c", subcore_axis_name="s")
    kern = pl.kernel(
        _mf_body,
        mesh=mesh,
        compiler_params=pltpu.CompilerParams(use_tc_tiling_on_sc=False,
                                             needs_layout_passes=False),
        out_type=jax.ShapeDtypeStruct((B,), jnp.float32),
        scratch_types=[
            pltpu.VMEM((NCHUNK, CH), jnp.int32),
            pltpu.VMEM((NCHUNK, CH), jnp.int32),
            pltpu.VMEM((BPW, D), jnp.float32),
            pltpu.VMEM((BPW, D), jnp.float32),
            pltpu.VMEM((BPW,), jnp.float32),
            pltpu.VMEM((BPW,), jnp.float32),
            pltpu.VMEM((BPW,), jnp.float32),
            pltpu.SemaphoreType.DMA,
        ],
    )
    return kern(user_table, item_table, ub_flat, ib_flat, user_ids, item_ids)


def kernel(user_table, item_table, user_bias_table, item_bias_table,
           global_bias, user_ids, item_ids):
    out = _mf(user_table, item_table,
              user_bias_table.reshape(-1), item_bias_table.reshape(-1),
              user_ids.astype(jnp.int32), item_ids.astype(jnp.int32))
    return out + global_bias[0]


# speedup vs baseline: 1.0131x; 1.0033x over previous
import jax
import jax.numpy as jnp
from jax import lax
from jax.experimental import pallas as pl
from jax.experimental.pallas import tpu as pltpu
from jax.experimental.pallas import tpu_sc as plsc

L = 16
NC = 2
NS = 16
NW = NC * NS
B = 16384
D = 64
BPW = B // NW
CH = 128
NCHUNK = BPW // CH
GROUPS = BPW // L


def _mf_body(ut, it, ubt, ibt, uid, iid, out,
             uid_v, iid_v, urows, irows, ub_v, ib_v, out_v, sem):
    wid = lax.axis_index("s") * NC + lax.axis_index("c")
    base = wid * BPW

    for j in range(NCHUNK):
        pltpu.sync_copy(uid.at[pl.ds(base + j * CH, CH)], uid_v.at[j])
        pltpu.sync_copy(iid.at[pl.ds(base + j * CH, CH)], iid_v.at[j])

    copies = []
    for j in range(NCHUNK):
        copies.append(pltpu.async_copy(
            ut.at[uid_v.at[j]], urows.at[pl.ds(j * CH, CH)], sem))
        copies.append(pltpu.async_copy(
            it.at[iid_v.at[j]], irows.at[pl.ds(j * CH, CH)], sem))
        copies.append(pltpu.async_copy(
            ubt.at[uid_v.at[j]], ub_v.at[pl.ds(j * CH, CH)], sem))
        copies.append(pltpu.async_copy(
            ibt.at[iid_v.at[j]], ib_v.at[pl.ds(j * CH, CH)], sem))
    for c in copies:
        c.wait()

    lane = lax.iota(jnp.int32, L)

    def group(g, carry):
        gb = pl.multiple_of(g * L, L)
        rows = gb + lane
        acc = ub_v[pl.ds(gb, L)] + ib_v[pl.ds(gb, L)]
        for k in range(D):
            col = lax.bitwise_and(lane + k, D - 1)
            u = plsc.load_gather(urows, [rows, col])
            v = plsc.load_gather(irows, [rows, col])
            acc = acc + u * v
        out_v[pl.ds(gb, L)] = acc
        return carry

    lax.fori_loop(0, GROUPS, group, 0)
    pltpu.sync_copy(out_v, out.at[pl.ds(base, BPW)])


@jax.jit
def _mf(user_table, item_table, ub_flat, ib_flat, user_ids, item_ids):
    mesh = plsc.VectorSubcoreMesh(core_axis_name="c", subcore_axis_name="s")
    kern = pl.kernel(
        _mf_body,
        mesh=mesh,
        compiler_params=pltpu.CompilerParams(use_tc_tiling_on_sc=False,
                                             needs_layout_passes=False),
        out_type=jax.ShapeDtypeStruct((B,), jnp.float32),
        scratch_types=[
            pltpu.VMEM((NCHUNK, CH), jnp.int32),
            pltpu.VMEM((NCHUNK, CH), jnp.int32),
            pltpu.VMEM((BPW, D), jnp.float32),
            pltpu.VMEM((BPW, D), jnp.float32),
            pltpu.VMEM((BPW,), jnp.float32),
            pltpu.VMEM((BPW,), jnp.float32),
            pltpu.VMEM((BPW,), jnp.float32),
            pltpu.SemaphoreType.DMA,
        ],
    )
    return kern(user_table, item_table, ub_flat, ib_flat, user_ids, item_ids)


def kernel(user_table, item_table, user_bias_table, item_bias_table,
           global_bias, user_ids, item_ids):
    out = _mf(user_table, item_table,
              user_bias_table[:, 0], item_bias_table[:, 0],
              user_ids.astype(jnp.int32), item_ids.astype(jnp.int32))
    return out + global_bias[0]


# speedup vs baseline: 1.3868x; 1.3689x over previous
import jax
import jax.numpy as jnp
from jax import lax
from jax.experimental import pallas as pl
from jax.experimental.pallas import tpu as pltpu
from jax.experimental.pallas import tpu_sc as plsc

L = 16
NC = 2
NS = 16
NW = NC * NS
B = 16384
D = 64
BPW = B // NW
CH = 128
NCHUNK = BPW // CH


def _mf_body(ut, it, ubt, ibt, uid, iid, out,
             uid_v, iid_v, uid2_v, iid2_v,
             ubuf, ibuf, ub_v, ib_v, out_v, sem):
    wid = lax.axis_index("s") * NC + lax.axis_index("c")
    base = wid * BPW

    pltpu.sync_copy(uid.at[pl.ds(base, BPW)], uid_v)
    pltpu.sync_copy(iid.at[pl.ds(base, BPW)], iid_v)
    for j in range(NCHUNK):
        pltpu.sync_copy(uid.at[pl.ds(base + j * CH, CH)], uid2_v.at[j])
        pltpu.sync_copy(iid.at[pl.ds(base + j * CH, CH)], iid2_v.at[j])

    bias_copies = []
    for j in range(NCHUNK):
        bias_copies.append(pltpu.async_copy(
            ubt.at[uid2_v.at[j]], ub_v.at[pl.ds(j * CH, CH)], sem))
        bias_copies.append(pltpu.async_copy(
            ibt.at[iid2_v.at[j]], ib_v.at[pl.ds(j * CH, CH)], sem))

    lane = lax.iota(jnp.int32, L)

    def chunk(c, carry):
        cb = pl.multiple_of(c * L, L)
        uids = uid_v[pl.ds(cb, L)]
        iids = iid_v[pl.ds(cb, L)]
        row_copies = []
        for s in range(L):
            row_copies.append(pltpu.async_copy(
                ut.at[uids[s]], ubuf.at[s], sem))
            row_copies.append(pltpu.async_copy(
                it.at[iids[s]], ibuf.at[s], sem))
        for rc in row_copies:
            rc.wait()
        acc = ub_v[pl.ds(cb, L)] + ib_v[pl.ds(cb, L)]
        for k in range(D):
            col = lax.bitwise_and(lane + k, D - 1)
            u = plsc.load_gather(ubuf, [lane, col])
            v = plsc.load_gather(ibuf, [lane, col])
            acc = acc + u * v
        out_v[pl.ds(cb, L)] = acc
        return carry

    for c in bias_copies:
        c.wait()
    lax.fori_loop(0, BPW // L, chunk, 0)
    pltpu.sync_copy(out_v, out.at[pl.ds(base, BPW)])


@jax.jit
def _mf(ut, it, ub_flat, ib_flat, uid, iid):
    mesh = plsc.VectorSubcoreMesh(core_axis_name="c", subcore_axis_name="s")
    kern = pl.kernel(
        _mf_body,
        mesh=mesh,
        compiler_params=pltpu.CompilerParams(needs_layout_passes=False),
        out_type=jax.ShapeDtypeStruct((B,), jnp.float32),
        scratch_types=[
            pltpu.VMEM((BPW,), jnp.int32),
            pltpu.VMEM((BPW,), jnp.int32),
            pltpu.VMEM((NCHUNK, CH), jnp.int32),
            pltpu.VMEM((NCHUNK, CH), jnp.int32),
            pltpu.VMEM((L, D), jnp.float32),
            pltpu.VMEM((L, D), jnp.float32),
            pltpu.VMEM((BPW,), jnp.float32),
            pltpu.VMEM((BPW,), jnp.float32),
            pltpu.VMEM((BPW,), jnp.float32),
            pltpu.SemaphoreType.DMA,
        ],
    )
    return kern(ut, it, ub_flat, ib_flat, uid, iid)


def kernel(user_table, item_table, user_bias_table, item_bias_table,
           global_bias, user_ids, item_ids):
    out = _mf(user_table, item_table,
              user_bias_table[:, 0], item_bias_table[:, 0],
              user_ids.astype(jnp.int32), item_ids.astype(jnp.int32))
    return out + global_bias[0]


# speedup vs baseline: 1.4065x; 1.0142x over previous
import jax
import jax.numpy as jnp
from jax import lax
from jax.experimental import pallas as pl
from jax.experimental.pallas import tpu as pltpu
from jax.experimental.pallas import tpu_sc as plsc

L = 16
NC = 2
NS = 16
NW = NC * NS
B = 16384
D = 64
BPW = B // NW
CH = 128
NCHUNK = BPW // CH


def _mf_body(ut, it, ubt, ibt, uid, iid, out,
             uid_v, iid_v, uid2_v, iid2_v,
             ubuf, ibuf, ub_v, ib_v, out_v, sem):
    wid = lax.axis_index("s") * NC + lax.axis_index("c")
    base = wid * BPW

    pltpu.sync_copy(uid.at[pl.ds(base, BPW)], uid_v)
    pltpu.sync_copy(iid.at[pl.ds(base, BPW)], iid_v)
    for j in range(NCHUNK):
        pltpu.sync_copy(uid.at[pl.ds(base + j * CH, CH)], uid2_v.at[j])
        pltpu.sync_copy(iid.at[pl.ds(base + j * CH, CH)], iid2_v.at[j])

    bias_copies = []
    for j in range(NCHUNK):
        bias_copies.append(pltpu.async_copy(
            ubt.at[uid2_v.at[j]], ub_v.at[pl.ds(j * CH, CH)], sem))
        bias_copies.append(pltpu.async_copy(
            ibt.at[iid2_v.at[j]], ib_v.at[pl.ds(j * CH, CH)], sem))

    lane = lax.iota(jnp.int32, L)

    def chunk(c, carry):
        fired = []
        for h in range(2):
            cb = pl.multiple_of(c * 2 * L + h * L, L)
            uids = uid_v[pl.ds(cb, L)]
            iids = iid_v[pl.ds(cb, L)]
            row_copies = []
            for s in range(L):
                row_copies.append(pltpu.async_copy(
                    ut.at[uids[s]], ubuf.at[h * L + s], sem))
                row_copies.append(pltpu.async_copy(
                    it.at[iids[s]], ibuf.at[h * L + s], sem))
            fired.append((cb, row_copies))
        for h in range(2):
            cb, row_copies = fired[h]
            for rc in row_copies:
                rc.wait()
            acc = ub_v[pl.ds(cb, L)] + ib_v[pl.ds(cb, L)]
            for k in range(D):
                col = lax.bitwise_and(lane + k, D - 1)
                u = plsc.load_gather(ubuf, [h * L + lane, col])
                v = plsc.load_gather(ibuf, [h * L + lane, col])
                acc = acc + u * v
            out_v[pl.ds(cb, L)] = acc
        return carry

    for c in bias_copies:
        c.wait()
    lax.fori_loop(0, BPW // (2 * L), chunk, 0)
    pltpu.sync_copy(out_v, out.at[pl.ds(base, BPW)])


@jax.jit
def _mf(ut, it, ub_flat, ib_flat, uid, iid):
    mesh = plsc.VectorSubcoreMesh(core_axis_name="c", subcore_axis_name="s")
    kern = pl.kernel(
        _mf_body,
        mesh=mesh,
        compiler_params=pltpu.CompilerParams(needs_layout_passes=False),
        out_type=jax.ShapeDtypeStruct((B,), jnp.float32),
        scratch_types=[
            pltpu.VMEM((BPW,), jnp.int32),
            pltpu.VMEM((BPW,), jnp.int32),
            pltpu.VMEM((NCHUNK, CH), jnp.int32),
            pltpu.VMEM((NCHUNK, CH), jnp.int32),
            pltpu.VMEM((2 * L, D), jnp.float32),
            pltpu.VMEM((2 * L, D), jnp.float32),
            pltpu.VMEM((BPW,), jnp.float32),
            pltpu.VMEM((BPW,), jnp.float32),
            pltpu.VMEM((BPW,), jnp.float32),
            pltpu.SemaphoreType.DMA,
        ],
    )
    return kern(ut, it, ub_flat, ib_flat, uid, iid)


def kernel(user_table, item_table, user_bias_table, item_bias_table,
           global_bias, user_ids, item_ids):
    out = _mf(user_table, item_table,
              user_bias_table[:, 0], item_bias_table[:, 0],
              user_ids.astype(jnp.int32), item_ids.astype(jnp.int32))
    return out + global_bias[0]
